# baseline (speedup 1.0000x reference)
import dataclasses
import functools

import jax
import jax.numpy as jnp
from jax import lax
from jax.experimental import pallas as pl
from jax.experimental.pallas import tpu as pltpu
from jax.experimental.pallas import tpu_sc as plsc

N = 10000
E = 160000
D = 128
G3 = 3 * D

NC = 2
NS = 16
NW = NC * NS
LANES = 16

CHUNK = 32
NCHUNKS = E // CHUNK
NJ = (NCHUNKS + NW - 1) // NW
ROWS_PER_SUB = N // NS
WROWS = 632

BE = 640
BN = 400

_HI = lax.Precision.HIGHEST
_f32 = jnp.float32


def _sig(a):
    return 1.0 / (1.0 + jnp.exp(-a))


def _gru_elem(gi, gh, h_prev):
    i_r, i_z, i_n = gi[:, :D], gi[:, D:2 * D], gi[:, 2 * D:]
    h_r, h_z, h_n = gh[:, :D], gh[:, D:2 * D], gh[:, 2 * D:]
    r = _sig(i_r + h_r)
    z = _sig(i_z + h_z)
    n = jnp.tanh(i_n + r * h_n)
    return (1.0 - z) * n + z * h_prev


def _gate_matmul(wg, h, bg):
    st = lax.dot_general(wg, h, (((1,), (1,)), ((), ())),
                         precision=_HI, preferred_element_type=_f32)
    return st + bg[:, 0:1]


def _stage0_body(inp_ref, wihT_ref, bih_ref, bhh_ref, wg_ref, bg_ref,
                 h_ref, st_ref):
    gi = jnp.dot(inp_ref[...], wihT_ref[...],
                 precision=_HI, preferred_element_type=_f32) + bih_ref[...]
    bhh = bhh_ref[...]
    i_r, i_z, i_n = gi[:, :D], gi[:, D:2 * D], gi[:, 2 * D:]
    h_r, h_z, h_n = bhh[:, :D], bhh[:, D:2 * D], bhh[:, 2 * D:]
    r = _sig(i_r + h_r)
    z = _sig(i_z + h_z)
    n = jnp.tanh(i_n + r * h_n)
    h = (1.0 - z) * n
    h_ref[...] = h
    st_ref[...] = _gate_matmul(wg_ref[...], h, bg_ref[...])


def _stage0_edge(inp, wihT, bih, bhh, wg, bg):
    grid = (E // BE,)
    return pl.pallas_call(
        _stage0_body,
        grid=grid,
        in_specs=[
            pl.BlockSpec((BE, D), lambda i: (i, 0)),
            pl.BlockSpec((D, G3), lambda i: (0, 0)),
            pl.BlockSpec((1, G3), lambda i: (0, 0)),
            pl.BlockSpec((1, G3), lambda i: (0, 0)),
            pl.BlockSpec((8, D), lambda i: (0, 0)),
            pl.BlockSpec((8, D), lambda i: (0, 0)),
        ],
        out_specs=[
            pl.BlockSpec((BE, D), lambda i: (i, 0)),
            pl.BlockSpec((8, BE), lambda i: (0, i)),
        ],
        out_shape=[
            jax.ShapeDtypeStruct((E, D), _f32),
            jax.ShapeDtypeStruct((8, E), _f32),
        ],
    )(inp, wihT, bih, bhh, wg, bg)


def _stage0_node_body(inp_ref, wihT_ref, bih_ref, bhh_ref, h_ref):
    gi = jnp.dot(inp_ref[...], wihT_ref[...],
                 precision=_HI, preferred_element_type=_f32) + bih_ref[...]
    bhh = bhh_ref[...]
    i_r, i_z, i_n = gi[:, :D], gi[:, D:2 * D], gi[:, 2 * D:]
    h_r, h_z, h_n = bhh[:, :D], bhh[:, D:2 * D], bhh[:, 2 * D:]
    r = _sig(i_r + h_r)
    z = _sig(i_z + h_z)
    n = jnp.tanh(i_n + r * h_n)
    h_ref[...] = (1.0 - z) * n


def _stage0_node(inp, wihT, bih, bhh):
    grid = (N // BN,)
    return pl.pallas_call(
        _stage0_node_body,
        grid=grid,
        in_specs=[
            pl.BlockSpec((BN, D), lambda i: (i, 0)),
            pl.BlockSpec((D, G3), lambda i: (0, 0)),
            pl.BlockSpec((1, G3), lambda i: (0, 0)),
            pl.BlockSpec((1, G3), lambda i: (0, 0)),
        ],
        out_specs=pl.BlockSpec((BN, D), lambda i: (i, 0)),
        out_shape=jax.ShapeDtypeStruct((N, D), _f32),
    )(inp, wihT, bih, bhh)


def _gate_st_body(h_ref, wg_ref, st_ref):
    st_ref[...] = lax.dot_general(h_ref[...], wg_ref[...],
                                  (((1,), (1,)), ((), ())),
                                  precision=_HI, preferred_element_type=_f32)


def _gate_st(h, wg):
    return pl.pallas_call(
        _gate_st_body,
        grid=(1,),
        in_specs=[
            pl.BlockSpec((N, D), lambda i: (0, 0)),
            pl.BlockSpec((8, D), lambda i: (0, 0)),
        ],
        out_specs=pl.BlockSpec((N, 8), lambda i: (0, 0)),
        out_shape=jax.ShapeDtypeStruct((N, 8), _f32),
    )(h, wg)


def _edge_layer_body(em_ref, efp_ref, wihT_ref, whhT_ref, bih_ref, bhh_ref,
                     wg_ref, bg_ref, h_ref, tt_ref):
    efp = efp_ref[...]
    gi = jnp.dot(em_ref[...], wihT_ref[...],
                 precision=_HI, preferred_element_type=_f32) + bih_ref[...]
    gh = jnp.dot(efp, whhT_ref[...],
                 precision=_HI, preferred_element_type=_f32) + bhh_ref[...]
    h = _gru_elem(gi, gh, efp)
    h_ref[...] = h
    tt_ref[...] = _gate_matmul(wg_ref[...], h, bg_ref[...])


def _edge_layer(em, efp, wihT, whhT, bih, bhh, wg, bg):
    grid = (E // BE,)
    return pl.pallas_call(
        _edge_layer_body,
        grid=grid,
        in_specs=[
            pl.BlockSpec((BE, D), lambda i: (i, 0)),
            pl.BlockSpec((BE, D), lambda i: (i, 0)),
            pl.BlockSpec((D, G3), lambda i: (0, 0)),
            pl.BlockSpec((D, G3), lambda i: (0, 0)),
            pl.BlockSpec((1, G3), lambda i: (0, 0)),
            pl.BlockSpec((1, G3), lambda i: (0, 0)),
            pl.BlockSpec((8, D), lambda i: (0, 0)),
            pl.BlockSpec((8, D), lambda i: (0, 0)),
        ],
        out_specs=[
            pl.BlockSpec((BE, D), lambda i: (i, 0)),
            pl.BlockSpec((8, BE), lambda i: (0, i)),
        ],
        out_shape=[
            jax.ShapeDtypeStruct((E, D), _f32),
            jax.ShapeDtypeStruct((8, E), _f32),
        ],
    )(em, efp, wihT, whhT, bih, bhh, wg, bg)


def _node_layer_body(s0_ref, s1_ref, c0_ref, c1_ref, xp_ref,
                     wihT_ref, whhT_ref, bih_ref, bhh_ref, h_ref):
    cnt = c0_ref[...][:, 0:1] + c1_ref[...][:, 0:1]
    msg = (s0_ref[...] + s1_ref[...]) / jnp.maximum(cnt, 1.0)
    xp = xp_ref[...]
    gi = jnp.dot(msg, wihT_ref[...],
                 precision=_HI, preferred_element_type=_f32) + bih_ref[...]
    gh = jnp.dot(xp, whhT_ref[...],
                 precision=_HI, preferred_element_type=_f32) + bhh_ref[...]
    h_ref[...] = _gru_elem(gi, gh, xp)


def _node_layer(sums, cnts, xp, wihT, whhT, bih, bhh):
    grid = (N // BN,)
    nblk = N // BN
    return pl.pallas_call(
        _node_layer_body,
        grid=grid,
        in_specs=[
            pl.BlockSpec((BN, D), lambda i: (i, 0)),
            pl.BlockSpec((BN, D), lambda i, n=nblk: (i + n, 0)),
            pl.BlockSpec((BN, 16), lambda i: (i, 0)),
            pl.BlockSpec((BN, 16), lambda i, n=nblk: (i + n, 0)),
            pl.BlockSpec((BN, D), lambda i: (i, 0)),
            pl.BlockSpec((D, G3), lambda i: (0, 0)),
            pl.BlockSpec((D, G3), lambda i: (0, 0)),
            pl.BlockSpec((1, G3), lambda i: (0, 0)),
            pl.BlockSpec((1, G3), lambda i: (0, 0)),
        ],
        out_specs=pl.BlockSpec((BN, D), lambda i: (i, 0)),
        out_shape=jax.ShapeDtypeStruct((N, D), _f32),
    )(sums, sums, cnts, cnts, xp, wihT, whhT, bih, bhh)


def _sc_compiler_params():
    cp = pltpu.CompilerParams()
    fields = pltpu.CompilerParams.__dataclass_fields__
    if "needs_layout_passes" in fields:
        cp = dataclasses.replace(cp, needs_layout_passes=False)
    if "use_tc_tiling_on_sc" in fields:
        cp = dataclasses.replace(cp, use_tc_tiling_on_sc=False)
    return cp


def _sc_stage(x, ef, src, dst, st, tt, zD, z16):
    mesh = plsc.VectorSubcoreMesh(core_axis_name="c", subcore_axis_name="s")

    @functools.partial(
        pl.kernel,
        compiler_params=_sc_compiler_params(),
        out_type=(
            jax.ShapeDtypeStruct((E, D), _f32),
            jax.ShapeDtypeStruct((2 * N, D), _f32),
            jax.ShapeDtypeStruct((2 * N, 16), _f32),
        ),
        mesh=mesh,
        scratch_types=[
            pltpu.VMEM((CHUNK,), jnp.int32),
            pltpu.VMEM((CHUNK,), jnp.int32),
            pltpu.VMEM((CHUNK, D), _f32),
            pltpu.VMEM((CHUNK, D), _f32),
            pltpu.VMEM((CHUNK, D), _f32),
            pltpu.VMEM((CHUNK, D), _f32),
            pltpu.VMEM((CHUNK, D), _f32),
            pltpu.VMEM((CHUNK, 8), _f32),
            pltpu.VMEM((CHUNK, 8), _f32),
            pltpu.VMEM((CHUNK,), _f32),
            pltpu.VMEM((CHUNK,), _f32),
            pltpu.VMEM((CHUNK,), _f32),
            pltpu.VMEM((CHUNK,), _f32),
            pltpu.VMEM((CHUNK,), _f32),
            pltpu.VMEM((CHUNK,), _f32),
            pltpu.VMEM((CHUNK,), _f32),
            pltpu.VMEM((CHUNK, 16), _f32),
            pltpu.VMEM_SHARED((N, D), _f32),
            pltpu.VMEM_SHARED((N, 16), _f32),
            pltpu.SemaphoreType.DMA,
            pltpu.SemaphoreType.DMA,
            pltpu.SemaphoreType.DMA,
            pltpu.SemaphoreType.DMA,
        ],
    )
    def k(x_hbm, ef_hbm, src_hbm, dst_hbm, st_hbm, tt_hbm, zD_hbm, z16_hbm,
          em_hbm, sums_hbm, cnt_hbm,
          dstv, srcv, xd, xs, efv, emv, nmv, sd, ss,
          t_sn, t_on, t_se, t_oe, gnb, ge1b, ge2b, onesb,
          sums_sp, cnt_sp, sem1, sem2, sem3, sem4):
        cid = lax.axis_index("c")
        sid = lax.axis_index("s")
        wid = sid * NC + cid
        zbase = pl.multiple_of((sid * ROWS_PER_SUB // 8) * 8, 8)

        @pl.loop(0, CHUNK)
        def _(i):
            onesb[i, :] = jnp.ones((LANES,), _f32)

        pltpu.sync_copy(zD_hbm.at[pl.ds(zbase, WROWS)],
                        sums_sp.at[pl.ds(zbase, WROWS)])
        pltpu.sync_copy(z16_hbm.at[pl.ds(zbase, WROWS)],
                        cnt_sp.at[pl.ds(zbase, WROWS)])
        plsc.subcore_barrier()

        @pl.loop(0, NJ)
        def _(j):
            ci_ = wid + NW * j

            @pl.when(ci_ < NCHUNKS)
            def _():
                base = ci_ * CHUNK
                pltpu.sync_copy(dst_hbm.at[pl.ds(base, CHUNK)], dstv)
                pltpu.sync_copy(src_hbm.at[pl.ds(base, CHUNK)], srcv)
                cp1 = pltpu.async_copy(x_hbm.at[dstv], xd, sem1)
                cp2 = pltpu.async_copy(x_hbm.at[srcv], xs, sem2)
                cp3 = pltpu.async_copy(st_hbm.at[dstv], sd, sem3)
                cp4 = pltpu.async_copy(st_hbm.at[srcv], ss, sem4)
                pltpu.sync_copy(ef_hbm.at[pl.ds(base, CHUNK)], efv)
                pltpu.sync_copy(tt_hbm.at[pl.ds(0 * E + base, CHUNK)], t_sn)
                pltpu.sync_copy(tt_hbm.at[pl.ds(1 * E + base, CHUNK)], t_on)
                pltpu.sync_copy(tt_hbm.at[pl.ds(2 * E + base, CHUNK)], t_se)
                pltpu.sync_copy(tt_hbm.at[pl.ds(3 * E + base, CHUNK)], t_oe)
                cp3.wait()
                cp4.wait()

                @pl.loop(0, CHUNK // LANES)
                def _(g):
                    gs = pl.ds(g * LANES, LANES)
                    rows = lax.iota(jnp.int32, LANES) + g * LANES
                    c0 = jnp.zeros((LANES,), jnp.int32)
                    a_sn = plsc.load_gather(sd, [rows, c0]) + t_sn[gs]
                    a_on = plsc.load_gather(ss, [rows, c0 + 1]) + t_on[gs]
                    a_se = plsc.load_gather(sd, [rows, c0 + 2]) + t_se[gs]
                    a_oe = plsc.load_gather(ss, [rows, c0 + 3]) + t_oe[gs]
                    gnb[gs] = _sig(a_sn) + _sig(a_on)
                    ge1b[gs] = _sig(a_se)
                    ge2b[gs] = _sig(a_oe)

                cp1.wait()
                cp2.wait()

                @pl.loop(0, CHUNK)
                def _(r):
                    ridx = jnp.zeros((LANES,), jnp.int32) + r
                    bn = plsc.load_gather(gnb, [ridx])
                    b1 = plsc.load_gather(ge1b, [ridx])
                    b2 = plsc.load_gather(ge2b, [ridx])
                    for c in range(D // LANES):
                        cs = pl.ds(c * LANES, LANES)
                        nmv[r, cs] = bn * efv[r, cs]
                        emv[r, cs] = b1 * xd[r, cs] + b2 * xs[r, cs]

                pltpu.sync_copy(nmv, sums_sp.at[dstv], add=True)
                pltpu.sync_copy(onesb, cnt_sp.at[dstv], add=True)
                pltpu.sync_copy(emv, em_hbm.at[pl.ds(base, CHUNK)])

        plsc.subcore_barrier()
        obase = pl.multiple_of(cid * N + zbase, 8)
        pltpu.sync_copy(sums_sp.at[pl.ds(zbase, WROWS)],
                        sums_hbm.at[pl.ds(obase, WROWS)])
        pltpu.sync_copy(cnt_sp.at[pl.ds(zbase, WROWS)],
                        cnt_hbm.at[pl.ds(obase, WROWS)])

    return k(x, ef, src, dst, st, tt, zD, z16)


def kernel(x, edge_feature, edge_index, Wih_node, Whh_node, bih_node,
           bhh_node, Wih_edge, Whh_edge, bih_edge, bhh_edge,
           W_sn, b_sn, W_on, b_on, W_se, b_se, W_oe, b_oe):
    src = edge_index[0]
    dst = edge_index[1]

    wihT_n, whhT_n = Wih_node.T, Whh_node.T
    wihT_e, whhT_e = Wih_edge.T, Whh_edge.T
    bih_n = bih_node.reshape(1, G3)
    bhh_n = bhh_node.reshape(1, G3)
    bih_e = bih_edge.reshape(1, G3)
    bhh_e = bhh_edge.reshape(1, G3)

    z4 = jnp.zeros((4, D), _f32)
    wgx = jnp.concatenate([W_sn[:, :D], W_on[:, :D], W_se[:, :D],
                           W_oe[:, :D], z4], axis=0)
    wge = jnp.concatenate([W_sn[:, D:], W_on[:, D:], W_se[:, D:],
                           W_oe[:, D:], z4], axis=0)
    bg = jnp.concatenate([b_sn, b_on, b_se, b_oe, jnp.zeros((4,), _f32)])
    bg_e = jnp.tile(bg[:, None], (1, D))
    bg_zero = jnp.zeros((8, D), _f32)

    zD = jnp.zeros((N, D), _f32)
    z16 = jnp.zeros((N, 16), _f32)

    x1 = _stage0_node(x, wihT_n, bih_n, bhh_n)
    st1 = _gate_st(x1, wgx)
    ef1, tt1 = _stage0_edge(edge_feature, wihT_e, bih_e, bhh_e, wge, bg_e)

    em1, sums1, cnt1 = _sc_stage(x1, ef1, src, dst, st1,
                                 tt1.reshape(-1), zD, z16)
    x2 = _node_layer(sums1, cnt1, x1, wihT_n, whhT_n, bih_n, bhh_n)
    st2 = _gate_st(x2, wgx)
    ef2, tt2 = _edge_layer(em1, ef1, wihT_e, whhT_e, bih_e, bhh_e, wge, bg_e)

    em2, sums2, cnt2 = _sc_stage(x2, ef2, src, dst, st2,
                                 tt2.reshape(-1), zD, z16)
    x3 = _node_layer(sums2, cnt2, x2, wihT_n, whhT_n, bih_n, bhh_n)
    ef3, _ = _edge_layer(em2, ef2, wihT_e, whhT_e, bih_e, bhh_e, wge, bg_e)

    return (x3, ef3)


# speedup vs baseline: 1.3900x; 1.3900x over previous
import dataclasses
import functools

import jax
import jax.numpy as jnp
from jax import lax
from jax.experimental import pallas as pl
from jax.experimental.pallas import tpu as pltpu
from jax.experimental.pallas import tpu_sc as plsc

N = 10000
E = 160000
D = 128
G3 = 3 * D

NC = 2
NS = 16
NW = NC * NS
LANES = 16

CHUNK = 32
NCHUNKS = E // CHUNK
NJ = (NCHUNKS + NW - 1) // NW
ROWS_PER_SUB = N // NS
WROWS = 632

BE = 640
BN = 400

_HI = lax.Precision.DEFAULT
_f32 = jnp.float32


def _sig(a):
    return 1.0 / (1.0 + jnp.exp(-a))


def _gru_elem(gi, gh, h_prev):
    i_r, i_z, i_n = gi[:, :D], gi[:, D:2 * D], gi[:, 2 * D:]
    h_r, h_z, h_n = gh[:, :D], gh[:, D:2 * D], gh[:, 2 * D:]
    r = _sig(i_r + h_r)
    z = _sig(i_z + h_z)
    n = jnp.tanh(i_n + r * h_n)
    return (1.0 - z) * n + z * h_prev


def _gate_matmul(wg, h, bg):
    st = lax.dot_general(wg, h, (((1,), (1,)), ((), ())),
                         precision=_HI, preferred_element_type=_f32)
    return st + bg[:, 0:1]


def _stage0_body(inp_ref, wihT_ref, bih_ref, bhh_ref, wg_ref, bg_ref,
                 h_ref, st_ref):
    gi = jnp.dot(inp_ref[...], wihT_ref[...],
                 precision=_HI, preferred_element_type=_f32) + bih_ref[...]
    bhh = bhh_ref[...]
    i_r, i_z, i_n = gi[:, :D], gi[:, D:2 * D], gi[:, 2 * D:]
    h_r, h_z, h_n = bhh[:, :D], bhh[:, D:2 * D], bhh[:, 2 * D:]
    r = _sig(i_r + h_r)
    z = _sig(i_z + h_z)
    n = jnp.tanh(i_n + r * h_n)
    h = (1.0 - z) * n
    h_ref[...] = h
    st_ref[...] = _gate_matmul(wg_ref[...], h, bg_ref[...])


def _stage0_edge(inp, wihT, bih, bhh, wg, bg):
    grid = (E // BE,)
    return pl.pallas_call(
        _stage0_body,
        grid=grid,
        in_specs=[
            pl.BlockSpec((BE, D), lambda i: (i, 0)),
            pl.BlockSpec((D, G3), lambda i: (0, 0)),
            pl.BlockSpec((1, G3), lambda i: (0, 0)),
            pl.BlockSpec((1, G3), lambda i: (0, 0)),
            pl.BlockSpec((8, D), lambda i: (0, 0)),
            pl.BlockSpec((8, D), lambda i: (0, 0)),
        ],
        out_specs=[
            pl.BlockSpec((BE, D), lambda i: (i, 0)),
            pl.BlockSpec((8, BE), lambda i: (0, i)),
        ],
        out_shape=[
            jax.ShapeDtypeStruct((E, D), _f32),
            jax.ShapeDtypeStruct((8, E), _f32),
        ],
    )(inp, wihT, bih, bhh, wg, bg)


def _stage0_node_body(inp_ref, wihT_ref, bih_ref, bhh_ref, h_ref):
    gi = jnp.dot(inp_ref[...], wihT_ref[...],
                 precision=_HI, preferred_element_type=_f32) + bih_ref[...]
    bhh = bhh_ref[...]
    i_r, i_z, i_n = gi[:, :D], gi[:, D:2 * D], gi[:, 2 * D:]
    h_r, h_z, h_n = bhh[:, :D], bhh[:, D:2 * D], bhh[:, 2 * D:]
    r = _sig(i_r + h_r)
    z = _sig(i_z + h_z)
    n = jnp.tanh(i_n + r * h_n)
    h_ref[...] = (1.0 - z) * n


def _stage0_node(inp, wihT, bih, bhh):
    grid = (N // BN,)
    return pl.pallas_call(
        _stage0_node_body,
        grid=grid,
        in_specs=[
            pl.BlockSpec((BN, D), lambda i: (i, 0)),
            pl.BlockSpec((D, G3), lambda i: (0, 0)),
            pl.BlockSpec((1, G3), lambda i: (0, 0)),
            pl.BlockSpec((1, G3), lambda i: (0, 0)),
        ],
        out_specs=pl.BlockSpec((BN, D), lambda i: (i, 0)),
        out_shape=jax.ShapeDtypeStruct((N, D), _f32),
    )(inp, wihT, bih, bhh)


def _gate_st_body(h_ref, wg_ref, st_ref):
    st_ref[...] = lax.dot_general(h_ref[...], wg_ref[...],
                                  (((1,), (1,)), ((), ())),
                                  precision=_HI, preferred_element_type=_f32)


def _gate_st(h, wg):
    return pl.pallas_call(
        _gate_st_body,
        grid=(1,),
        in_specs=[
            pl.BlockSpec((N, D), lambda i: (0, 0)),
            pl.BlockSpec((8, D), lambda i: (0, 0)),
        ],
        out_specs=pl.BlockSpec((N, 8), lambda i: (0, 0)),
        out_shape=jax.ShapeDtypeStruct((N, 8), _f32),
    )(h, wg)


def _edge_layer_body(em_ref, efp_ref, wihT_ref, whhT_ref, bih_ref, bhh_ref,
                     wg_ref, bg_ref, h_ref, tt_ref):
    efp = efp_ref[...]
    gi = jnp.dot(em_ref[...], wihT_ref[...],
                 precision=_HI, preferred_element_type=_f32) + bih_ref[...]
    gh = jnp.dot(efp, whhT_ref[...],
                 precision=_HI, preferred_element_type=_f32) + bhh_ref[...]
    h = _gru_elem(gi, gh, efp)
    h_ref[...] = h
    tt_ref[...] = _gate_matmul(wg_ref[...], h, bg_ref[...])


def _edge_layer(em, efp, wihT, whhT, bih, bhh, wg, bg):
    grid = (E // BE,)
    return pl.pallas_call(
        _edge_layer_body,
        grid=grid,
        in_specs=[
            pl.BlockSpec((BE, D), lambda i: (i, 0)),
            pl.BlockSpec((BE, D), lambda i: (i, 0)),
            pl.BlockSpec((D, G3), lambda i: (0, 0)),
            pl.BlockSpec((D, G3), lambda i: (0, 0)),
            pl.BlockSpec((1, G3), lambda i: (0, 0)),
            pl.BlockSpec((1, G3), lambda i: (0, 0)),
            pl.BlockSpec((8, D), lambda i: (0, 0)),
            pl.BlockSpec((8, D), lambda i: (0, 0)),
        ],
        out_specs=[
            pl.BlockSpec((BE, D), lambda i: (i, 0)),
            pl.BlockSpec((8, BE), lambda i: (0, i)),
        ],
        out_shape=[
            jax.ShapeDtypeStruct((E, D), _f32),
            jax.ShapeDtypeStruct((8, E), _f32),
        ],
    )(em, efp, wihT, whhT, bih, bhh, wg, bg)


def _node_layer_body(s0_ref, s1_ref, c0_ref, c1_ref, xp_ref,
                     wihT_ref, whhT_ref, bih_ref, bhh_ref, h_ref):
    cnt = c0_ref[...][:, 0:1] + c1_ref[...][:, 0:1]
    msg = (s0_ref[...] + s1_ref[...]) / jnp.maximum(cnt, 1.0)
    xp = xp_ref[...]
    gi = jnp.dot(msg, wihT_ref[...],
                 precision=_HI, preferred_element_type=_f32) + bih_ref[...]
    gh = jnp.dot(xp, whhT_ref[...],
                 precision=_HI, preferred_element_type=_f32) + bhh_ref[...]
    h_ref[...] = _gru_elem(gi, gh, xp)


def _node_layer(sums, cnts, xp, wihT, whhT, bih, bhh):
    grid = (N // BN,)
    nblk = N // BN
    return pl.pallas_call(
        _node_layer_body,
        grid=grid,
        in_specs=[
            pl.BlockSpec((BN, D), lambda i: (i, 0)),
            pl.BlockSpec((BN, D), lambda i, n=nblk: (i + n, 0)),
            pl.BlockSpec((BN, 16), lambda i: (i, 0)),
            pl.BlockSpec((BN, 16), lambda i, n=nblk: (i + n, 0)),
            pl.BlockSpec((BN, D), lambda i: (i, 0)),
            pl.BlockSpec((D, G3), lambda i: (0, 0)),
            pl.BlockSpec((D, G3), lambda i: (0, 0)),
            pl.BlockSpec((1, G3), lambda i: (0, 0)),
            pl.BlockSpec((1, G3), lambda i: (0, 0)),
        ],
        out_specs=pl.BlockSpec((BN, D), lambda i: (i, 0)),
        out_shape=jax.ShapeDtypeStruct((N, D), _f32),
    )(sums, sums, cnts, cnts, xp, wihT, whhT, bih, bhh)


def _sc_compiler_params():
    cp = pltpu.CompilerParams()
    fields = pltpu.CompilerParams.__dataclass_fields__
    if "needs_layout_passes" in fields:
        cp = dataclasses.replace(cp, needs_layout_passes=False)
    if "use_tc_tiling_on_sc" in fields:
        cp = dataclasses.replace(cp, use_tc_tiling_on_sc=False)
    return cp


def _sc_stage(x, ef, src, dst, st, tt, zD, z16):
    mesh = plsc.VectorSubcoreMesh(core_axis_name="c", subcore_axis_name="s")

    @functools.partial(
        pl.kernel,
        compiler_params=_sc_compiler_params(),
        out_type=(
            jax.ShapeDtypeStruct((E, D), _f32),
            jax.ShapeDtypeStruct((2 * N, D), _f32),
            jax.ShapeDtypeStruct((2 * N, 16), _f32),
        ),
        mesh=mesh,
        scratch_types=[
            pltpu.VMEM((CHUNK,), jnp.int32),
            pltpu.VMEM((CHUNK,), jnp.int32),
            pltpu.VMEM((CHUNK, D), _f32),
            pltpu.VMEM((CHUNK, D), _f32),
            pltpu.VMEM((CHUNK, D), _f32),
            pltpu.VMEM((CHUNK, D), _f32),
            pltpu.VMEM((CHUNK, D), _f32),
            pltpu.VMEM((CHUNK, 8), _f32),
            pltpu.VMEM((CHUNK, 8), _f32),
            pltpu.VMEM((CHUNK,), _f32),
            pltpu.VMEM((CHUNK,), _f32),
            pltpu.VMEM((CHUNK,), _f32),
            pltpu.VMEM((CHUNK,), _f32),
            pltpu.VMEM((CHUNK,), _f32),
            pltpu.VMEM((CHUNK,), _f32),
            pltpu.VMEM((CHUNK,), _f32),
            pltpu.VMEM((CHUNK, 16), _f32),
            pltpu.VMEM_SHARED((N, D), _f32),
            pltpu.VMEM_SHARED((N, 16), _f32),
            pltpu.SemaphoreType.DMA,
            pltpu.SemaphoreType.DMA,
            pltpu.SemaphoreType.DMA,
            pltpu.SemaphoreType.DMA,
        ],
    )
    def k(x_hbm, ef_hbm, src_hbm, dst_hbm, st_hbm, tt_hbm, zD_hbm, z16_hbm,
          em_hbm, sums_hbm, cnt_hbm,
          dstv, srcv, xd, xs, efv, emv, nmv, sd, ss,
          t_sn, t_on, t_se, t_oe, gnb, ge1b, ge2b, onesb,
          sums_sp, cnt_sp, sem1, sem2, sem3, sem4):
        cid = lax.axis_index("c")
        sid = lax.axis_index("s")
        wid = sid * NC + cid
        zbase = pl.multiple_of((sid * ROWS_PER_SUB // 8) * 8, 8)

        @pl.loop(0, CHUNK)
        def _(i):
            onesb[i, :] = jnp.ones((LANES,), _f32)

        pltpu.sync_copy(zD_hbm.at[pl.ds(zbase, WROWS)],
                        sums_sp.at[pl.ds(zbase, WROWS)])
        pltpu.sync_copy(z16_hbm.at[pl.ds(zbase, WROWS)],
                        cnt_sp.at[pl.ds(zbase, WROWS)])
        plsc.subcore_barrier()

        @pl.loop(0, NJ)
        def _(j):
            ci_ = wid + NW * j

            @pl.when(ci_ < NCHUNKS)
            def _():
                base = ci_ * CHUNK
                pltpu.sync_copy(dst_hbm.at[pl.ds(base, CHUNK)], dstv)
                pltpu.sync_copy(src_hbm.at[pl.ds(base, CHUNK)], srcv)
                cp1 = pltpu.async_copy(x_hbm.at[dstv], xd, sem1)
                cp2 = pltpu.async_copy(x_hbm.at[srcv], xs, sem2)
                cp3 = pltpu.async_copy(st_hbm.at[dstv], sd, sem3)
                cp4 = pltpu.async_copy(st_hbm.at[srcv], ss, sem4)
                pltpu.sync_copy(ef_hbm.at[pl.ds(base, CHUNK)], efv)
                pltpu.sync_copy(tt_hbm.at[pl.ds(0 * E + base, CHUNK)], t_sn)
                pltpu.sync_copy(tt_hbm.at[pl.ds(1 * E + base, CHUNK)], t_on)
                pltpu.sync_copy(tt_hbm.at[pl.ds(2 * E + base, CHUNK)], t_se)
                pltpu.sync_copy(tt_hbm.at[pl.ds(3 * E + base, CHUNK)], t_oe)
                cp3.wait()
                cp4.wait()

                @pl.loop(0, CHUNK // LANES)
                def _(g):
                    gs = pl.ds(g * LANES, LANES)
                    rows = lax.iota(jnp.int32, LANES) + g * LANES
                    c0 = jnp.zeros((LANES,), jnp.int32)
                    a_sn = plsc.load_gather(sd, [rows, c0]) + t_sn[gs]
                    a_on = plsc.load_gather(ss, [rows, c0 + 1]) + t_on[gs]
                    a_se = plsc.load_gather(sd, [rows, c0 + 2]) + t_se[gs]
                    a_oe = plsc.load_gather(ss, [rows, c0 + 3]) + t_oe[gs]
                    gnb[gs] = _sig(a_sn) + _sig(a_on)
                    ge1b[gs] = _sig(a_se)
                    ge2b[gs] = _sig(a_oe)

                cp1.wait()
                cp2.wait()

                @pl.loop(0, CHUNK)
                def _(r):
                    ridx = jnp.zeros((LANES,), jnp.int32) + r
                    bn = plsc.load_gather(gnb, [ridx])
                    b1 = plsc.load_gather(ge1b, [ridx])
                    b2 = plsc.load_gather(ge2b, [ridx])
                    for c in range(D // LANES):
                        cs = pl.ds(c * LANES, LANES)
                        nmv[r, cs] = bn * efv[r, cs]
                        emv[r, cs] = b1 * xd[r, cs] + b2 * xs[r, cs]

                pltpu.sync_copy(nmv, sums_sp.at[dstv], add=True)
                pltpu.sync_copy(onesb, cnt_sp.at[dstv], add=True)
                pltpu.sync_copy(emv, em_hbm.at[pl.ds(base, CHUNK)])

        plsc.subcore_barrier()
        obase = pl.multiple_of(cid * N + zbase, 8)
        pltpu.sync_copy(sums_sp.at[pl.ds(zbase, WROWS)],
                        sums_hbm.at[pl.ds(obase, WROWS)])
        pltpu.sync_copy(cnt_sp.at[pl.ds(zbase, WROWS)],
                        cnt_hbm.at[pl.ds(obase, WROWS)])

    return k(x, ef, src, dst, st, tt, zD, z16)


def kernel(x, edge_feature, edge_index, Wih_node, Whh_node, bih_node,
           bhh_node, Wih_edge, Whh_edge, bih_edge, bhh_edge,
           W_sn, b_sn, W_on, b_on, W_se, b_se, W_oe, b_oe):
    src = edge_index[0]
    dst = edge_index[1]

    wihT_n, whhT_n = Wih_node.T, Whh_node.T
    wihT_e, whhT_e = Wih_edge.T, Whh_edge.T
    bih_n = bih_node.reshape(1, G3)
    bhh_n = bhh_node.reshape(1, G3)
    bih_e = bih_edge.reshape(1, G3)
    bhh_e = bhh_edge.reshape(1, G3)

    z4 = jnp.zeros((4, D), _f32)
    wgx = jnp.concatenate([W_sn[:, :D], W_on[:, :D], W_se[:, :D],
                           W_oe[:, :D], z4], axis=0)
    wge = jnp.concatenate([W_sn[:, D:], W_on[:, D:], W_se[:, D:],
                           W_oe[:, D:], z4], axis=0)
    bg = jnp.concatenate([b_sn, b_on, b_se, b_oe, jnp.zeros((4,), _f32)])
    bg_e = jnp.tile(bg[:, None], (1, D))
    bg_zero = jnp.zeros((8, D), _f32)

    zD = jnp.zeros((N, D), _f32)
    z16 = jnp.zeros((N, 16), _f32)

    x1 = _stage0_node(x, wihT_n, bih_n, bhh_n)
    st1 = _gate_st(x1, wgx)
    ef1, tt1 = _stage0_edge(edge_feature, wihT_e, bih_e, bhh_e, wge, bg_e)

    em1, sums1, cnt1 = _sc_stage(x1, ef1, src, dst, st1,
                                 tt1.reshape(-1), zD, z16)
    x2 = _node_layer(sums1, cnt1, x1, wihT_n, whhT_n, bih_n, bhh_n)
    st2 = _gate_st(x2, wgx)
    ef2, tt2 = _edge_layer(em1, ef1, wihT_e, whhT_e, bih_e, bhh_e, wge, bg_e)

    em2, sums2, cnt2 = _sc_stage(x2, ef2, src, dst, st2,
                                 tt2.reshape(-1), zD, z16)
    x3 = _node_layer(sums2, cnt2, x2, wihT_n, whhT_n, bih_n, bhh_n)
    ef3, _ = _edge_layer(em2, ef2, wihT_e, whhT_e, bih_e, bhh_e, wge, bg_e)

    return (x3, ef3)


# speedup vs baseline: 1.9860x; 1.4288x over previous
import dataclasses
import functools

import jax
import jax.numpy as jnp
from jax import lax
from jax.experimental import pallas as pl
from jax.experimental.pallas import tpu as pltpu
from jax.experimental.pallas import tpu_sc as plsc

N = 10000
E = 160000
D = 128
G3 = 3 * D

NC = 2
NS = 16
NW = NC * NS
LANES = 16

CHUNK = 32
NCHUNKS = E // CHUNK
NJ = (NCHUNKS + NW - 1) // NW
NJP = (NJ + 1) // 2
CHUNKC = 128
NCHC = E // CHUNKC
NJC = (NCHC + NW - 1) // NW
ROWS_PER_SUB = N // NS
WROWS = 632

BE = 640
BN = 400

_HI = lax.Precision.DEFAULT
_f32 = jnp.float32


def _sig(a):
    return 1.0 / (1.0 + jnp.exp(-a))


def _gru_elem(gi, gh, h_prev):
    i_r, i_z, i_n = gi[:, :D], gi[:, D:2 * D], gi[:, 2 * D:]
    h_r, h_z, h_n = gh[:, :D], gh[:, D:2 * D], gh[:, 2 * D:]
    r = _sig(i_r + h_r)
    z = _sig(i_z + h_z)
    n = jnp.tanh(i_n + r * h_n)
    return (1.0 - z) * n + z * h_prev


def _gate_rows(h, wg, bg):
    return lax.dot_general(h, wg, (((1,), (1,)), ((), ())),
                           precision=_HI, preferred_element_type=_f32) + bg


def _stage0_body(inp_ref, wihT_ref, bih_ref, bhh_ref, wg_ref, bg_ref,
                 h_ref, tt_ref):
    gi = jnp.dot(inp_ref[...], wihT_ref[...],
                 precision=_HI, preferred_element_type=_f32) + bih_ref[...]
    bhh = bhh_ref[...]
    i_r, i_z, i_n = gi[:, :D], gi[:, D:2 * D], gi[:, 2 * D:]
    h_r, h_z, h_n = bhh[:, :D], bhh[:, D:2 * D], bhh[:, 2 * D:]
    r = _sig(i_r + h_r)
    z = _sig(i_z + h_z)
    n = jnp.tanh(i_n + r * h_n)
    h = (1.0 - z) * n
    h_ref[...] = h
    tt_ref[...] = _gate_rows(h, wg_ref[...], bg_ref[...])


def _stage0_edge(inp, wihT, bih, bhh, wg, bg):
    grid = (E // BE,)
    return pl.pallas_call(
        _stage0_body,
        grid=grid,
        in_specs=[
            pl.BlockSpec((BE, D), lambda i: (i, 0)),
            pl.BlockSpec((D, G3), lambda i: (0, 0)),
            pl.BlockSpec((1, G3), lambda i: (0, 0)),
            pl.BlockSpec((1, G3), lambda i: (0, 0)),
            pl.BlockSpec((8, D), lambda i: (0, 0)),
            pl.BlockSpec((1, 8), lambda i: (0, 0)),
        ],
        out_specs=[
            pl.BlockSpec((BE, D), lambda i: (i, 0)),
            pl.BlockSpec((BE, 8), lambda i: (i, 0)),
        ],
        out_shape=[
            jax.ShapeDtypeStruct((E, D), _f32),
            jax.ShapeDtypeStruct((E, 8), _f32),
        ],
    )(inp, wihT, bih, bhh, wg, bg)


def _stage0_node_body(inp_ref, wihT_ref, bih_ref, bhh_ref, h_ref):
    gi = jnp.dot(inp_ref[...], wihT_ref[...],
                 precision=_HI, preferred_element_type=_f32) + bih_ref[...]
    bhh = bhh_ref[...]
    i_r, i_z, i_n = gi[:, :D], gi[:, D:2 * D], gi[:, 2 * D:]
    h_r, h_z, h_n = bhh[:, :D], bhh[:, D:2 * D], bhh[:, 2 * D:]
    r = _sig(i_r + h_r)
    z = _sig(i_z + h_z)
    n = jnp.tanh(i_n + r * h_n)
    h_ref[...] = (1.0 - z) * n


def _stage0_node(inp, wihT, bih, bhh):
    grid = (N // BN,)
    return pl.pallas_call(
        _stage0_node_body,
        grid=grid,
        in_specs=[
            pl.BlockSpec((BN, D), lambda i: (i, 0)),
            pl.BlockSpec((D, G3), lambda i: (0, 0)),
            pl.BlockSpec((1, G3), lambda i: (0, 0)),
            pl.BlockSpec((1, G3), lambda i: (0, 0)),
        ],
        out_specs=pl.BlockSpec((BN, D), lambda i: (i, 0)),
        out_shape=jax.ShapeDtypeStruct((N, D), _f32),
    )(inp, wihT, bih, bhh)


def _gate_st_body(h_ref, wg_ref, st_ref):
    st_ref[...] = lax.dot_general(h_ref[...], wg_ref[...],
                                  (((1,), (1,)), ((), ())),
                                  precision=_HI, preferred_element_type=_f32)


def _gate_st(h, wg):
    return pl.pallas_call(
        _gate_st_body,
        grid=(1,),
        in_specs=[
            pl.BlockSpec((N, D), lambda i: (0, 0)),
            pl.BlockSpec((8, D), lambda i: (0, 0)),
        ],
        out_specs=pl.BlockSpec((N, 8), lambda i: (0, 0)),
        out_shape=jax.ShapeDtypeStruct((N, 8), _f32),
    )(h, wg)


def _edge_layer_body(em_ref, efp_ref, wihT_ref, whhT_ref, bih_ref, bhh_ref,
                     wg_ref, bg_ref, h_ref, tt_ref):
    efp = efp_ref[...]
    gi = jnp.dot(em_ref[...], wihT_ref[...],
                 precision=_HI, preferred_element_type=_f32) + bih_ref[...]
    gh = jnp.dot(efp, whhT_ref[...],
                 precision=_HI, preferred_element_type=_f32) + bhh_ref[...]
    h = _gru_elem(gi, gh, efp)
    h_ref[...] = h
    tt_ref[...] = _gate_rows(h, wg_ref[...], bg_ref[...])


def _edge_layer(em, efp, wihT, whhT, bih, bhh, wg, bg):
    grid = (E // BE,)
    return pl.pallas_call(
        _edge_layer_body,
        grid=grid,
        in_specs=[
            pl.BlockSpec((BE, D), lambda i: (i, 0)),
            pl.BlockSpec((BE, D), lambda i: (i, 0)),
            pl.BlockSpec((D, G3), lambda i: (0, 0)),
            pl.BlockSpec((D, G3), lambda i: (0, 0)),
            pl.BlockSpec((1, G3), lambda i: (0, 0)),
            pl.BlockSpec((1, G3), lambda i: (0, 0)),
            pl.BlockSpec((8, D), lambda i: (0, 0)),
            pl.BlockSpec((1, 8), lambda i: (0, 0)),
        ],
        out_specs=[
            pl.BlockSpec((BE, D), lambda i: (i, 0)),
            pl.BlockSpec((BE, 8), lambda i: (i, 0)),
        ],
        out_shape=[
            jax.ShapeDtypeStruct((E, D), _f32),
            jax.ShapeDtypeStruct((E, 8), _f32),
        ],
    )(em, efp, wihT, whhT, bih, bhh, wg, bg)


def _node_layer_body(s0_ref, s1_ref, c0_ref, c1_ref, xp_ref,
                     wihT_ref, whhT_ref, bih_ref, bhh_ref, h_ref):
    cnt = c0_ref[...][:, 0:1] + c1_ref[...][:, 0:1]
    msg = (s0_ref[...] + s1_ref[...]) / jnp.maximum(cnt, 1.0)
    xp = xp_ref[...]
    gi = jnp.dot(msg, wihT_ref[...],
                 precision=_HI, preferred_element_type=_f32) + bih_ref[...]
    gh = jnp.dot(xp, whhT_ref[...],
                 precision=_HI, preferred_element_type=_f32) + bhh_ref[...]
    h_ref[...] = _gru_elem(gi, gh, xp)


def _node_layer(sums, cnts, xp, wihT, whhT, bih, bhh):
    grid = (N // BN,)
    nblk = N // BN
    return pl.pallas_call(
        _node_layer_body,
        grid=grid,
        in_specs=[
            pl.BlockSpec((BN, D), lambda i: (i, 0)),
            pl.BlockSpec((BN, D), lambda i, n=nblk: (i + n, 0)),
            pl.BlockSpec((BN, 8), lambda i: (i, 0)),
            pl.BlockSpec((BN, 8), lambda i, n=nblk: (i + n, 0)),
            pl.BlockSpec((BN, D), lambda i: (i, 0)),
            pl.BlockSpec((D, G3), lambda i: (0, 0)),
            pl.BlockSpec((D, G3), lambda i: (0, 0)),
            pl.BlockSpec((1, G3), lambda i: (0, 0)),
            pl.BlockSpec((1, G3), lambda i: (0, 0)),
        ],
        out_specs=pl.BlockSpec((BN, D), lambda i: (i, 0)),
        out_shape=jax.ShapeDtypeStruct((N, D), _f32),
    )(sums, sums, cnts, cnts, xp, wihT, whhT, bih, bhh)


def _sc_compiler_params():
    cp = pltpu.CompilerParams()
    fields = pltpu.CompilerParams.__dataclass_fields__
    if "needs_layout_passes" in fields:
        cp = dataclasses.replace(cp, needs_layout_passes=False)
    if "use_tc_tiling_on_sc" in fields:
        cp = dataclasses.replace(cp, use_tc_tiling_on_sc=False)
    return cp


def _sc_counts(dst, ones8, z8):
    mesh = plsc.VectorSubcoreMesh(core_axis_name="c", subcore_axis_name="s")

    @functools.partial(
        pl.kernel,
        compiler_params=_sc_compiler_params(),
        out_type=jax.ShapeDtypeStruct((2 * N, 8), _f32),
        mesh=mesh,
        scratch_types=[
            pltpu.VMEM((CHUNKC,), jnp.int32),
            pltpu.VMEM((CHUNKC, 8), _f32),
            pltpu.VMEM_SHARED((N, 8), _f32),
        ],
    )
    def k(dst_hbm, ones_hbm, z8_hbm, cnt_hbm, dstc, onesv, cnt_sp):
        cid = lax.axis_index("c")
        sid = lax.axis_index("s")
        wid = sid * NC + cid
        zbase = pl.multiple_of((sid * ROWS_PER_SUB // 8) * 8, 8)

        pltpu.sync_copy(z8_hbm.at[pl.ds(zbase, WROWS)],
                        cnt_sp.at[pl.ds(zbase, WROWS)])
        pltpu.sync_copy(ones_hbm, onesv)
        plsc.subcore_barrier()

        @pl.loop(0, NJC)
        def _(j):
            ci = wid + NW * j

            @pl.when(ci < NCHC)
            def _():
                base = ci * CHUNKC
                pltpu.sync_copy(dst_hbm.at[pl.ds(base, CHUNKC)], dstc)
                pltpu.sync_copy(onesv, cnt_sp.at[dstc], add=True)

        plsc.subcore_barrier()
        obase = pl.multiple_of(cid * N + zbase, 8)
        pltpu.sync_copy(cnt_sp.at[pl.ds(zbase, WROWS)],
                        cnt_hbm.at[pl.ds(obase, WROWS)])

    return k(dst, ones8, z8)


def _sc_stage(x, ef, src, dst, st, te, zD):
    mesh = plsc.VectorSubcoreMesh(core_axis_name="c", subcore_axis_name="s")

    @functools.partial(
        pl.kernel,
        compiler_params=_sc_compiler_params(),
        out_type=(
            jax.ShapeDtypeStruct((E, D), _f32),
            jax.ShapeDtypeStruct((2 * N, D), _f32),
        ),
        mesh=mesh,
        scratch_types=[
            pltpu.VMEM((CHUNK,), jnp.int32),
            pltpu.VMEM((CHUNK,), jnp.int32),
            pltpu.VMEM((CHUNK,), jnp.int32),
            pltpu.VMEM((CHUNK,), jnp.int32),
            pltpu.VMEM((CHUNK,), jnp.int32),
            pltpu.VMEM((CHUNK,), jnp.int32),
            pltpu.VMEM((CHUNK, D), _f32),
            pltpu.VMEM((CHUNK, D), _f32),
            pltpu.VMEM((CHUNK, D), _f32),
            pltpu.VMEM((CHUNK, D), _f32),
            pltpu.VMEM((CHUNK, D), _f32),
            pltpu.VMEM((CHUNK, D), _f32),
            pltpu.VMEM((CHUNK, 8), _f32),
            pltpu.VMEM((CHUNK, 8), _f32),
            pltpu.VMEM((CHUNK, 8), _f32),
            pltpu.VMEM((CHUNK, 8), _f32),
            pltpu.VMEM((CHUNK, 8), _f32),
            pltpu.VMEM((CHUNK, 8), _f32),
            pltpu.VMEM((CHUNK, D), _f32),
            pltpu.VMEM((CHUNK, D), _f32),
            pltpu.VMEM((CHUNK, D), _f32),
            pltpu.VMEM((CHUNK, D), _f32),
            pltpu.VMEM((CHUNK,), _f32),
            pltpu.VMEM((CHUNK,), _f32),
            pltpu.VMEM((CHUNK,), _f32),
            pltpu.VMEM_SHARED((N, D), _f32),
            pltpu.SemaphoreType.DMA,
            pltpu.SemaphoreType.DMA,
            pltpu.SemaphoreType.DMA,
            pltpu.SemaphoreType.DMA,
            pltpu.SemaphoreType.DMA,
            pltpu.SemaphoreType.DMA,
        ],
    )
    def k(x_hbm, ef_hbm, src_hbm, dst_hbm, st_hbm, te_hbm, zD_hbm,
          em_hbm, sums_hbm,
          dstv0, srcv0, dstv1, srcv1, dstc0, dstc1,
          xd0, xs0, efv0, xd1, xs1, efv1,
          sd0, ss0, tv0, sd1, ss1, tv1,
          emv0, nmv0, emv1, nmv1,
          gnb, ge1b, ge2b, sums_sp,
          sem_ix0, sem_ix1, sem_in0, sem_in1, sem_out0, sem_out1):
        cid = lax.axis_index("c")
        sid = lax.axis_index("s")
        wid = sid * NC + cid
        zbase = pl.multiple_of((sid * ROWS_PER_SUB // 8) * 8, 8)

        slots = (
            (dstv0, srcv0, dstc0, xd0, xs0, efv0, sd0, ss0, tv0, emv0, nmv0,
             sem_ix0, sem_in0, sem_out0),
            (dstv1, srcv1, dstc1, xd1, xs1, efv1, sd1, ss1, tv1, emv1, nmv1,
             sem_ix1, sem_in1, sem_out1),
        )

        def valid(j):
            ci = wid + NW * j
            return ci, (ci >= 0) & (ci < NCHUNKS)

        def idx_load(j, s):
            dstv, srcv = s[0], s[1]
            sem_ix = s[11]
            ci, ok = valid(j)

            @pl.when(ok)
            def _():
                base = ci * CHUNK
                pltpu.async_copy(dst_hbm.at[pl.ds(base, CHUNK)], dstv, sem_ix)
                pltpu.async_copy(src_hbm.at[pl.ds(base, CHUNK)], srcv, sem_ix)

        def bulk_load(j, s):
            dstv, srcv, _, xd, xs, efv, sd, ss, tv = s[:9]
            sem_ix, sem_in = s[11], s[12]
            ci, ok = valid(j)

            @pl.when(ok)
            def _():
                base = ci * CHUNK
                pltpu.make_async_copy(dst_hbm.at[pl.ds(base, CHUNK)], dstv,
                                      sem_ix).wait()
                pltpu.make_async_copy(src_hbm.at[pl.ds(base, CHUNK)], srcv,
                                      sem_ix).wait()
                pltpu.async_copy(x_hbm.at[dstv], xd, sem_in)
                pltpu.async_copy(x_hbm.at[srcv], xs, sem_in)
                pltpu.async_copy(st_hbm.at[dstv], sd, sem_in)
                pltpu.async_copy(st_hbm.at[srcv], ss, sem_in)
                pltpu.async_copy(ef_hbm.at[pl.ds(base, CHUNK)], efv, sem_in)
                pltpu.async_copy(te_hbm.at[pl.ds(base, CHUNK)], tv, sem_in)

        def wait_out(j, s):
            emv = s[9]
            sem_out = s[13]
            ci, ok = valid(j)

            @pl.when(ok)
            def _():
                base = ci * CHUNK
                pltpu.make_async_copy(emv, em_hbm.at[pl.ds(base, CHUNK)],
                                      sem_out).wait()

        def proc(j, s):
            dstv, srcv, dstc, xd, xs, efv, sd, ss, tv, emv, nmv = s[:11]
            sem_in, sem_out = s[12], s[13]
            ci, ok = valid(j)

            @pl.when(ok)
            def _():
                base = ci * CHUNK
                pltpu.make_async_copy(x_hbm.at[pl.ds(0, CHUNK)], xd,
                                      sem_in).wait()
                pltpu.make_async_copy(x_hbm.at[pl.ds(0, CHUNK)], xs,
                                      sem_in).wait()
                pltpu.make_async_copy(st_hbm.at[pl.ds(0, CHUNK)], sd,
                                      sem_in).wait()
                pltpu.make_async_copy(st_hbm.at[pl.ds(0, CHUNK)], ss,
                                      sem_in).wait()
                pltpu.make_async_copy(ef_hbm.at[pl.ds(base, CHUNK)], efv,
                                      sem_in).wait()
                pltpu.make_async_copy(te_hbm.at[pl.ds(base, CHUNK)], tv,
                                      sem_in).wait()

                @pl.loop(0, CHUNK // LANES)
                def _(g):
                    gs = pl.ds(g * LANES, LANES)
                    rows = lax.iota(jnp.int32, LANES) + g * LANES
                    c0 = jnp.zeros((LANES,), jnp.int32)
                    dstc[gs] = dstv[gs]
                    a_sn = (plsc.load_gather(sd, [rows, c0])
                            + plsc.load_gather(tv, [rows, c0]))
                    a_on = (plsc.load_gather(ss, [rows, c0 + 1])
                            + plsc.load_gather(tv, [rows, c0 + 1]))
                    a_se = (plsc.load_gather(sd, [rows, c0 + 2])
                            + plsc.load_gather(tv, [rows, c0 + 2]))
                    a_oe = (plsc.load_gather(ss, [rows, c0 + 3])
                            + plsc.load_gather(tv, [rows, c0 + 3]))
                    gnb[gs] = _sig(a_sn) + _sig(a_on)
                    ge1b[gs] = _sig(a_se)
                    ge2b[gs] = _sig(a_oe)

                @pl.loop(0, CHUNK)
                def _(r):
                    ridx = jnp.zeros((LANES,), jnp.int32) + r
                    bn = plsc.load_gather(gnb, [ridx])
                    b1 = plsc.load_gather(ge1b, [ridx])
                    b2 = plsc.load_gather(ge2b, [ridx])
                    for c in range(D // LANES):
                        cs = pl.ds(c * LANES, LANES)
                        nmv[r, cs] = bn * efv[r, cs]
                        emv[r, cs] = b1 * xd[r, cs] + b2 * xs[r, cs]

                pltpu.sync_copy(nmv, sums_sp.at[dstc], add=True)
                pltpu.async_copy(emv, em_hbm.at[pl.ds(base, CHUNK)], sem_out)

        pltpu.sync_copy(zD_hbm.at[pl.ds(zbase, WROWS)],
                        sums_sp.at[pl.ds(zbase, WROWS)])
        plsc.subcore_barrier()

        idx_load(0, slots[0])
        bulk_load(0, slots[0])
        idx_load(1, slots[1])

        @pl.loop(0, NJP)
        def _(jj):
            j0 = jj * 2
            j1 = j0 + 1
            bulk_load(j1, slots[1])
            wait_out(j0 - 2, slots[0])
            proc(j0, slots[0])
            idx_load(j0 + 2, slots[0])
            bulk_load(j0 + 2, slots[0])
            wait_out(j1 - 2, slots[1])
            proc(j1, slots[1])
            idx_load(j1 + 2, slots[1])

        wait_out(2 * NJP - 2, slots[0])
        wait_out(2 * NJP - 1, slots[1])

        plsc.subcore_barrier()
        obase = pl.multiple_of(cid * N + zbase, 8)
        pltpu.sync_copy(sums_sp.at[pl.ds(zbase, WROWS)],
                        sums_hbm.at[pl.ds(obase, WROWS)])

    return k(x, ef, src, dst, st, te, zD)


def kernel(x, edge_feature, edge_index, Wih_node, Whh_node, bih_node,
           bhh_node, Wih_edge, Whh_edge, bih_edge, bhh_edge,
           W_sn, b_sn, W_on, b_on, W_se, b_se, W_oe, b_oe):
    src = edge_index[0]
    dst = edge_index[1]

    wihT_n, whhT_n = Wih_node.T, Whh_node.T
    wihT_e, whhT_e = Wih_edge.T, Whh_edge.T
    bih_n = bih_node.reshape(1, G3)
    bhh_n = bhh_node.reshape(1, G3)
    bih_e = bih_edge.reshape(1, G3)
    bhh_e = bhh_edge.reshape(1, G3)

    z4 = jnp.zeros((4, D), _f32)
    wgx = jnp.concatenate([W_sn[:, :D], W_on[:, :D], W_se[:, :D],
                           W_oe[:, :D], z4], axis=0)
    wge = jnp.concatenate([W_sn[:, D:], W_on[:, D:], W_se[:, D:],
                           W_oe[:, D:], z4], axis=0)
    bg_e = jnp.concatenate([b_sn, b_on, b_se, b_oe,
                            jnp.zeros((4,), _f32)]).reshape(1, 8)

    zD = jnp.zeros((N, D), _f32)
    z8 = jnp.zeros((N, 8), _f32)
    ones8 = jnp.ones((CHUNKC, 8), _f32)

    cnt = _sc_counts(dst, ones8, z8)

    x1 = _stage0_node(x, wihT_n, bih_n, bhh_n)
    st1 = _gate_st(x1, wgx)
    ef1, tt1 = _stage0_edge(edge_feature, wihT_e, bih_e, bhh_e, wge, bg_e)

    em1, sums1 = _sc_stage(x1, ef1, src, dst, st1, tt1, zD)
    x2 = _node_layer(sums1, cnt, x1, wihT_n, whhT_n, bih_n, bhh_n)
    st2 = _gate_st(x2, wgx)
    ef2, tt2 = _edge_layer(em1, ef1, wihT_e, whhT_e, bih_e, bhh_e, wge, bg_e)

    em2, sums2 = _sc_stage(x2, ef2, src, dst, st2, tt2, zD)
    x3 = _node_layer(sums2, cnt, x2, wihT_n, whhT_n, bih_n, bhh_n)
    ef3, _ = _edge_layer(em2, ef2, wihT_e, whhT_e, bih_e, bhh_e, wge, bg_e)

    return (x3, ef3)
